# baseline (speedup 1.0000x reference)
import functools

import jax
import jax.numpy as jnp
from jax import lax
from jax.experimental import pallas as pl
from jax.experimental.pallas import tpu as pltpu
from jax.experimental.pallas import tpu_sc as plsc

BATCH = 16384
EMBED_DIM = 16
HIDDEN_DIM = 64
NC = 2
NS = 16
NW = NC * NS
B_PER_W = BATCH // NW
STREAM = 128
N_CHUNK = B_PER_W // STREAM


def _sc_gather(user_table, item_table, uids, iids):
    mesh = plsc.VectorSubcoreMesh(core_axis_name="c", subcore_axis_name="s")

    @functools.partial(
        pl.kernel,
        mesh=mesh,
        compiler_params=pltpu.CompilerParams(use_tc_tiling_on_sc=False),
        out_type=[
            jax.ShapeDtypeStruct((BATCH, EMBED_DIM), jnp.float32),
            jax.ShapeDtypeStruct((BATCH, EMBED_DIM), jnp.float32),
        ],
        scratch_types=[
            pltpu.VMEM((N_CHUNK, STREAM), jnp.int32),
            pltpu.VMEM((N_CHUNK, STREAM), jnp.int32),
            pltpu.VMEM((B_PER_W, EMBED_DIM), jnp.float32),
            pltpu.VMEM((B_PER_W, EMBED_DIM), jnp.float32),
            pltpu.SemaphoreType.DMA,
            pltpu.SemaphoreType.DMA,
        ],
    )
    def k(utab_hbm, itab_hbm, uid_hbm, iid_hbm, uout_hbm, iout_hbm,
          uidx_v, iidx_v, urows_v, irows_v, usem, isem):
        wid = lax.axis_index("s") * NC + lax.axis_index("c")
        base = wid * B_PER_W
        pltpu.sync_copy(uid_hbm.at[wid], uidx_v)
        pltpu.sync_copy(iid_hbm.at[wid], iidx_v)
        copies = []
        for c in range(N_CHUNK):
            copies.append(pltpu.async_copy(
                utab_hbm.at[uidx_v.at[c]],
                urows_v.at[pl.ds(c * STREAM, STREAM)], usem))
            copies.append(pltpu.async_copy(
                itab_hbm.at[iidx_v.at[c]],
                irows_v.at[pl.ds(c * STREAM, STREAM)], isem))
        for cp in copies:
            cp.wait()
        pltpu.sync_copy(urows_v, uout_hbm.at[pl.ds(base, B_PER_W)])
        pltpu.sync_copy(irows_v, iout_hbm.at[pl.ds(base, B_PER_W)])

    return k(user_table, item_table, uids, iids)


def _mlp_body(ue_ref, ie_ref, w1u_ref, w1i_ref, b1_ref, w2_ref, b2_ref,
              w3_ref, b3_ref, out_ref):
    h1 = jnp.dot(ue_ref[...], w1u_ref[...], preferred_element_type=jnp.float32)
    h1 += jnp.dot(ie_ref[...], w1i_ref[...], preferred_element_type=jnp.float32)
    h1 = jax.nn.relu(h1 + b1_ref[...])
    h2 = jax.nn.relu(
        jnp.dot(h1, w2_ref[...], preferred_element_type=jnp.float32)
        + b2_ref[...])
    o = jnp.sum(h2 * w3_ref[...], axis=1, keepdims=True) + b3_ref[...]
    out_ref[...] = jax.nn.sigmoid(o)


def _tc_mlp(ue, ie, W1, b1, W2, b2, W3, b3):
    blk = 2048
    grid = (BATCH // blk,)
    w1u = W1[:, :EMBED_DIM].T
    w1i = W1[:, EMBED_DIM:].T
    w2 = W2.T
    b1r = b1.reshape(1, HIDDEN_DIM)
    b2r = b2.reshape(1, HIDDEN_DIM)
    w3r = W3.reshape(1, HIDDEN_DIM)
    b3r = b3.reshape(1, 1)
    full = lambda shape: pl.BlockSpec(shape, lambda i: (0, 0))
    return pl.pallas_call(
        _mlp_body,
        grid=grid,
        in_specs=[
            pl.BlockSpec((blk, EMBED_DIM), lambda i: (i, 0)),
            pl.BlockSpec((blk, EMBED_DIM), lambda i: (i, 0)),
            full((EMBED_DIM, HIDDEN_DIM)),
            full((EMBED_DIM, HIDDEN_DIM)),
            full((1, HIDDEN_DIM)),
            full((HIDDEN_DIM, HIDDEN_DIM)),
            full((1, HIDDEN_DIM)),
            full((1, HIDDEN_DIM)),
            full((1, 1)),
        ],
        out_specs=pl.BlockSpec((blk, 1), lambda i: (i, 0)),
        out_shape=jax.ShapeDtypeStruct((BATCH, 1), jnp.float32),
    )(ue, ie, w1u, w1i, b1r, w2, b2r, w3r, b3r)


def kernel(user_ids, item_ids, user_table, item_table, W1, b1, W2, b2, W3, b3):
    uids = user_ids.astype(jnp.int32).reshape(NW, N_CHUNK, STREAM)
    iids = item_ids.astype(jnp.int32).reshape(NW, N_CHUNK, STREAM)
    ue, ie = _sc_gather(user_table, item_table, uids, iids)
    return _tc_mlp(ue, ie, W1, b1, W2, b2, W3, b3)
